# Initial kernel scaffold; baseline (speedup 1.0000x reference)
import jax
import jax.numpy as jnp
from jax.experimental import pallas as pl


def kernel(input, T, adj, edge, p, weight, bias):
    raise NotImplementedError("write your pallas kernel here")


# speedup vs baseline: 1.9115x; 1.9115x over previous
import functools

import jax
import jax.numpy as jnp
from jax import lax
from jax.experimental import pallas as pl
from jax.experimental.pallas import tpu as pltpu
from jax.experimental.pallas import tpu_sc as plsc

N = 4096
E = 16384
IN_F = 256
OUT_F = 128

def _k_support(p_ref, x_ref, w_ref, sm_ref, prod_ref):
    p2 = 2.0 * jax.nn.sigmoid(p_ref[...])
    support = jnp.dot(x_ref[...], w_ref[...], preferred_element_type=jnp.float32)
    e = support * p2
    sm = jnp.exp(e - jnp.max(e))
    sm_ref[...] = sm
    prod_ref[...] = support * sm


def _support_sm_prod(p, x, w):
    return pl.pallas_call(
        _k_support,
        out_shape=(
            jax.ShapeDtypeStruct((N, OUT_F), jnp.float32),
            jax.ShapeDtypeStruct((N, OUT_F), jnp.float32),
        ),
    )(p, x, w)


_RB2 = 512

def _k_agg(adj_ref, sm_ref, agg_ref):
    agg_ref[...] = jnp.dot(adj_ref[...], sm_ref[...],
                           preferred_element_type=jnp.float32)


def _agg(adj, sm):
    grid = (N // _RB2,)
    return pl.pallas_call(
        _k_agg,
        grid=grid,
        in_specs=[
            pl.BlockSpec((_RB2, N), lambda i: (i, 0)),
            pl.BlockSpec((N, OUT_F), lambda i: (0, 0)),
        ],
        out_specs=pl.BlockSpec((_RB2, OUT_F), lambda i: (i, 0)),
        out_shape=jax.ShapeDtypeStruct((N, OUT_F), jnp.float32),
        compiler_params=pltpu.CompilerParams(
            dimension_semantics=("arbitrary",)),
    )(adj, sm)


_NC = 2
_NS = 16
_NW = _NC * _NS
_EPW = E // _NW
_CHUNK = 128
_NCHUNK = _EPW // _CHUNK


def _gather_rows(prod, agg, e1, e0):
    mesh = plsc.VectorSubcoreMesh(core_axis_name="c", subcore_axis_name="s")

    @functools.partial(
        pl.kernel,
        mesh=mesh,
        out_type=(
            jax.ShapeDtypeStruct((E, OUT_F), jnp.float32),
            jax.ShapeDtypeStruct((E, OUT_F), jnp.float32),
        ),
        scratch_types=[
            pltpu.VMEM((_CHUNK,), jnp.int32),
            pltpu.VMEM((_CHUNK,), jnp.int32),
            pltpu.VMEM((_CHUNK, OUT_F), jnp.float32),
            pltpu.VMEM((_CHUNK, OUT_F), jnp.float32),
            pltpu.SemaphoreType.DMA,
            pltpu.SemaphoreType.DMA,
        ],
    )
    def k(prod_hbm, agg_hbm, e1_hbm, e0_hbm, gp_hbm, ga_hbm,
          idx1_v, idx0_v, rows1_v, rows0_v, sem1, sem0):
        wid = lax.axis_index("s") * _NC + lax.axis_index("c")
        base = wid * _EPW
        for c in range(_NCHUNK):
            off = base + c * _CHUNK
            pltpu.sync_copy(e1_hbm.at[pl.ds(off, _CHUNK)], idx1_v)
            pltpu.sync_copy(e0_hbm.at[pl.ds(off, _CHUNK)], idx0_v)
            cp1 = pltpu.async_copy(prod_hbm.at[idx1_v], rows1_v, sem1)
            cp0 = pltpu.async_copy(agg_hbm.at[idx0_v], rows0_v, sem0)
            cp1.wait()
            cp0.wait()
            pltpu.sync_copy(rows1_v, gp_hbm.at[pl.ds(off, _CHUNK)])
            pltpu.sync_copy(rows0_v, ga_hbm.at[pl.ds(off, _CHUNK)])

    return k(prod, agg, e1, e0)


_KB = 1024

def _k_out(t_ref, gp_ref, ga_ref, b_ref, out_ref):
    j = pl.program_id(0)
    msg = gp_ref[...] / (ga_ref[...] + 1e-6)
    part = jnp.dot(t_ref[...], msg, preferred_element_type=jnp.float32)

    @pl.when(j == 0)
    def _():
        out_ref[...] = part + b_ref[...]

    @pl.when(j > 0)
    def _():
        out_ref[...] = out_ref[...] + part


def _final(T, gp, ga, bias):
    grid = (E // _KB,)
    return pl.pallas_call(
        _k_out,
        grid=grid,
        in_specs=[
            pl.BlockSpec((N, _KB), lambda j: (0, j)),
            pl.BlockSpec((_KB, OUT_F), lambda j: (j, 0)),
            pl.BlockSpec((_KB, OUT_F), lambda j: (j, 0)),
            pl.BlockSpec((1, OUT_F), lambda j: (0, 0)),
        ],
        out_specs=pl.BlockSpec((N, OUT_F), lambda j: (0, 0)),
        out_shape=jax.ShapeDtypeStruct((N, OUT_F), jnp.float32),
        compiler_params=pltpu.CompilerParams(
            dimension_semantics=("arbitrary",)),
    )(T, gp, ga, bias)


def kernel(input, T, adj, edge, p, weight, bias):
    p11 = p.reshape(1, 1)
    sm, prod = _support_sm_prod(p11, input, weight)
    agg = _agg(adj, sm)
    e1 = edge[1]
    e0 = edge[0]
    gp, ga = _gather_rows(prod, agg, e1, e0)
    return _final(T, gp, ga, bias.reshape(1, OUT_F))
